# baseline (speedup 1.0000x reference)
import functools

import jax
import jax.numpy as jnp
from jax import lax
from jax.experimental import pallas as pl
from jax.experimental.pallas import tpu as pltpu
from jax.experimental.pallas import tpu_sc as plsc

N, C, H, W = 4, 96, 224, 224
HW = H * W
B = N * HW
NW = 32
BPW = B // NW
CHUNK = 112
NCHUNK = BPW // CHUNK
LANES = 16
GRPS = CHUNK // LANES
CV = C // LANES
WPB = HW // BPW


def _floor_i32(v):
    t = v.astype(jnp.int32)
    return t - jnp.where(t.astype(jnp.float32) > v, 1, 0)


ROWS = N * H // NW


def _sc_transpose(inp):
    mesh = plsc.VectorSubcoreMesh(core_axis_name="c", subcore_axis_name="s")

    buf = lambda shape, dt: [pltpu.VMEM(shape, dt) for _ in range(2)]

    @functools.partial(
        pl.kernel,
        mesh=mesh,
        out_type=jax.ShapeDtypeStruct((B, C), jnp.float32),
        compiler_params=pltpu.CompilerParams(
            needs_layout_passes=False, use_tc_tiling_on_sc=False),
        scratch_types=[
            buf((C, W), jnp.float32),
            buf((W, C), jnp.float32),
            [pltpu.SemaphoreType.DMA for _ in range(2)],
            [pltpu.SemaphoreType.DMA for _ in range(2)],
        ],
    )
    def kt(inp_hbm, table_hbm, inbuf, tbuf, semi, semo):
        wid = lax.axis_index("s") * 2 + lax.axis_index("c")
        rowbase = wid * ROWS
        lanes = lax.iota(jnp.int32, LANES)

        def issue_in(j, s):
            gr = rowbase + j
            n = gr // H
            h = gr % H
            pltpu.async_copy(inp_hbm.at[n, :, h, :], inbuf[s], semi[s])

        def drain_in(s):
            pltpu.make_async_copy(
                inp_hbm.at[0, :, 0, :], inbuf[s], semi[s]).wait()

        def transpose(s):
            @pl.loop(0, W, unroll=4)
            def _(pix):
                widx = lax.broadcast(pix, (LANES,))
                for c0 in range(CV):
                    v = plsc.load_gather(inbuf[s], [lanes + c0 * LANES, widx])
                    tbuf[s][pix, pl.ds(c0 * LANES, LANES)] = v

        def issue_out(j, s):
            gr = rowbase + j
            pltpu.async_copy(tbuf[s], table_hbm.at[pl.ds(gr * W, W)], semo[s])

        def drain_out(s):
            pltpu.make_async_copy(
                tbuf[s], table_hbm.at[pl.ds(0, W)], semo[s]).wait()

        issue_in(0, 0)

        @pl.loop(0, ROWS, step=2)
        def _(g):
            for b in (0, 1):
                cur = g + b

                @pl.when(cur + 1 < ROWS)
                def _():
                    issue_in(cur + 1, 1 - b)

                drain_in(b)

                @pl.when(cur >= 2)
                def _():
                    drain_out(b)

                transpose(b)
                issue_out(cur, b)

        drain_out(0)
        drain_out(1)

    return kt(inp)


def _sc_grid_sample(table, gx, gy):
    mesh = plsc.VectorSubcoreMesh(core_axis_name="c", subcore_axis_name="s")

    buf = lambda shape, dt: [pltpu.VMEM(shape, dt) for _ in range(2)]

    @functools.partial(
        pl.kernel,
        mesh=mesh,
        out_type=jax.ShapeDtypeStruct((N, C, H, W), jnp.float32),
        compiler_params=pltpu.CompilerParams(
            needs_layout_passes=False, use_tc_tiling_on_sc=False),
        scratch_types=[
            buf((CHUNK,), jnp.float32),
            buf((CHUNK,), jnp.float32),
            buf((CHUNK,), jnp.int32),
            buf((CHUNK,), jnp.int32),
            buf((CHUNK,), jnp.int32),
            buf((CHUNK,), jnp.int32),
            buf((CHUNK,), jnp.float32),
            buf((CHUNK,), jnp.float32),
            buf((CHUNK,), jnp.float32),
            buf((CHUNK,), jnp.float32),
            buf((CHUNK, C), jnp.float32),
            buf((CHUNK, C), jnp.float32),
            buf((CHUNK, C), jnp.float32),
            buf((CHUNK, C), jnp.float32),
            buf((C, CHUNK), jnp.float32),
            [pltpu.SemaphoreType.DMA for _ in range(2)],
        ],
    )
    def k(table_hbm, gx_hbm, gy_hbm, out_hbm,
          gxv, gyv, i00, i01, i10, i11, w00, w01, w10, w11,
          r00, r01, r10, r11, outv, sem):
        wid = lax.axis_index("s") * 2 + lax.axis_index("c")
        base = wid * BPW
        nidx = wid // WPB
        nbase = nidx * HW
        qbase = (wid % WPB) * BPW

        def stage(cur, s):
            off = base + cur * CHUNK
            pltpu.sync_copy(gx_hbm.at[pl.ds(off, CHUNK)], gxv[s])
            pltpu.sync_copy(gy_hbm.at[pl.ds(off, CHUNK)], gyv[s])

            def grp_body(j, c2):
                sl = pl.ds(j * LANES, LANES)
                x = gxv[s][sl]
                y = gyv[s][sl]
                ix = (x + 1.0) * (W * 0.5) - 0.5
                iy = (y + 1.0) * (H * 0.5) - 0.5
                x0 = _floor_i32(ix)
                y0 = _floor_i32(iy)
                fx = ix - x0.astype(jnp.float32)
                fy = iy - y0.astype(jnp.float32)
                vx0 = (x0 >= 0) & (x0 <= W - 1)
                vx1 = (x0 >= -1) & (x0 <= W - 2)
                vy0 = (y0 >= 0) & (y0 <= H - 1)
                vy1 = (y0 >= -1) & (y0 <= H - 2)
                zero = jnp.zeros((LANES,), jnp.float32)
                wx0 = jnp.where(vx0, 1.0 - fx, zero)
                wx1 = jnp.where(vx1, fx, zero)
                wy0 = jnp.where(vy0, 1.0 - fy, zero)
                wy1 = jnp.where(vy1, fy, zero)
                xc0 = jnp.clip(x0, 0, W - 1)
                xc1 = jnp.clip(x0 + 1, 0, W - 1)
                yr0 = nbase + jnp.clip(y0, 0, H - 1) * W
                yr1 = nbase + jnp.clip(y0 + 1, 0, H - 1) * W
                i00[s][sl] = yr0 + xc0
                i01[s][sl] = yr0 + xc1
                i10[s][sl] = yr1 + xc0
                i11[s][sl] = yr1 + xc1
                w00[s][sl] = wy0 * wx0
                w01[s][sl] = wy0 * wx1
                w10[s][sl] = wy1 * wx0
                w11[s][sl] = wy1 * wx1
                return c2

            lax.fori_loop(0, GRPS, grp_body, 0)

            pltpu.async_copy(table_hbm.at[i00[s]], r00[s], sem[s])
            pltpu.async_copy(table_hbm.at[i01[s]], r01[s], sem[s])
            pltpu.async_copy(table_hbm.at[i10[s]], r10[s], sem[s])
            pltpu.async_copy(table_hbm.at[i11[s]], r11[s], sem[s])

        def drain(s):
            pltpu.make_async_copy(table_hbm.at[i00[s]], r00[s], sem[s]).wait()
            pltpu.make_async_copy(table_hbm.at[i01[s]], r01[s], sem[s]).wait()
            pltpu.make_async_copy(table_hbm.at[i10[s]], r10[s], sem[s]).wait()
            pltpu.make_async_copy(table_hbm.at[i11[s]], r11[s], sem[s]).wait()

        def interp(cur, s):
            q = qbase + cur * CHUNK
            hrow = q // W
            wcol = q % W
            lanes = lax.iota(jnp.int32, LANES)

            def pix_body(i, c2):
                widx = lax.broadcast(i, (LANES,))
                s00 = plsc.load_gather(w00[s], [widx])
                s01 = plsc.load_gather(w01[s], [widx])
                s10 = plsc.load_gather(w10[s], [widx])
                s11 = plsc.load_gather(w11[s], [widx])
                for r in range(CV):
                    cs = r * LANES
                    a = r00[s][i, pl.ds(cs, LANES)] * s00
                    a = a + r01[s][i, pl.ds(cs, LANES)] * s01
                    a = a + r10[s][i, pl.ds(cs, LANES)] * s10
                    a = a + r11[s][i, pl.ds(cs, LANES)] * s11
                    plsc.store_scatter(outv[s], [lanes + cs, widx], a)
                return c2

            lax.fori_loop(0, CHUNK, pix_body, 0)
            pltpu.sync_copy(outv[s],
                            out_hbm.at[nidx, :, hrow, pl.ds(wcol, CHUNK)])

        stage(0, 0)

        @pl.loop(0, NCHUNK, step=2)
        def _(g):
            for b in (0, 1):
                cur = g + b

                @pl.when(cur + 1 < NCHUNK)
                def _():
                    stage(cur + 1, 1 - b)

                drain(b)
                interp(cur, b)

    return k(table, gx, gy)


def kernel(input, grid):
    table = _sc_transpose(input)
    gx = grid[..., 0].reshape(B)
    gy = grid[..., 1].reshape(B)
    return _sc_grid_sample(table, gx, gy)


# speedup vs baseline: 1.2659x; 1.2659x over previous
import functools

import jax
import jax.numpy as jnp
from jax import lax
from jax.experimental import pallas as pl
from jax.experimental.pallas import tpu as pltpu
from jax.experimental.pallas import tpu_sc as plsc

N, C, H, W = 4, 96, 224, 224
HW = H * W
B = N * HW
NW = 32
BPW = B // NW
CHUNK = 112
NCHUNK = BPW // CHUNK
LANES = 16
GRPS = CHUNK // LANES
CV = C // LANES
WPB = HW // BPW


def _floor_i32(v):
    t = v.astype(jnp.int32)
    return t - jnp.where(t.astype(jnp.float32) > v, 1, 0)


ROWS = N * H // NW


def _sc_transpose(inp):
    mesh = plsc.VectorSubcoreMesh(core_axis_name="c", subcore_axis_name="s")

    buf = lambda shape, dt: [pltpu.VMEM(shape, dt) for _ in range(2)]

    @functools.partial(
        pl.kernel,
        mesh=mesh,
        out_type=jax.ShapeDtypeStruct((B, C), jnp.float32),
        compiler_params=pltpu.CompilerParams(
            needs_layout_passes=False, use_tc_tiling_on_sc=False),
        scratch_types=[
            buf((C, W + 1), jnp.float32),
            buf((W, C), jnp.float32),
            [pltpu.SemaphoreType.DMA for _ in range(2)],
            [pltpu.SemaphoreType.DMA for _ in range(2)],
        ],
    )
    def kt(inp_hbm, table_hbm, inbuf, tbuf, semi, semo):
        wid = lax.axis_index("s") * 2 + lax.axis_index("c")
        rowbase = wid * ROWS
        lanes = lax.iota(jnp.int32, LANES)

        def issue_in(j, s):
            gr = rowbase + j
            n = gr // H
            h = gr % H
            pltpu.async_copy(
                inp_hbm.at[n, :, h, :], inbuf[s].at[:, pl.ds(0, W)], semi[s])

        def drain_in(s):
            pltpu.make_async_copy(
                inp_hbm.at[0, :, 0, :], inbuf[s].at[:, pl.ds(0, W)],
                semi[s]).wait()

        def transpose(s):
            @pl.loop(0, W, unroll=4)
            def _(pix):
                widx = lax.broadcast(pix, (LANES,))
                for c0 in range(CV):
                    v = plsc.load_gather(inbuf[s], [lanes + c0 * LANES, widx])
                    tbuf[s][pix, pl.ds(c0 * LANES, LANES)] = v

        def issue_out(j, s):
            gr = rowbase + j
            pltpu.async_copy(tbuf[s], table_hbm.at[pl.ds(gr * W, W)], semo[s])

        def drain_out(s):
            pltpu.make_async_copy(
                tbuf[s], table_hbm.at[pl.ds(0, W)], semo[s]).wait()

        issue_in(0, 0)

        @pl.loop(0, ROWS, step=2)
        def _(g):
            for b in (0, 1):
                cur = g + b

                @pl.when(cur + 1 < ROWS)
                def _():
                    issue_in(cur + 1, 1 - b)

                drain_in(b)

                @pl.when(cur >= 2)
                def _():
                    drain_out(b)

                transpose(b)
                issue_out(cur, b)

        drain_out(0)
        drain_out(1)

    return kt(inp)


def _sc_grid_sample(table, gx, gy):
    mesh = plsc.VectorSubcoreMesh(core_axis_name="c", subcore_axis_name="s")

    buf = lambda shape, dt: [pltpu.VMEM(shape, dt) for _ in range(2)]

    @functools.partial(
        pl.kernel,
        mesh=mesh,
        out_type=jax.ShapeDtypeStruct((N, C, H, W), jnp.float32),
        compiler_params=pltpu.CompilerParams(
            needs_layout_passes=False, use_tc_tiling_on_sc=False),
        scratch_types=[
            buf((CHUNK,), jnp.float32),
            buf((CHUNK,), jnp.float32),
            buf((CHUNK,), jnp.int32),
            buf((CHUNK,), jnp.int32),
            buf((CHUNK,), jnp.int32),
            buf((CHUNK,), jnp.int32),
            buf((CHUNK,), jnp.float32),
            buf((CHUNK,), jnp.float32),
            buf((CHUNK,), jnp.float32),
            buf((CHUNK,), jnp.float32),
            buf((CHUNK, C), jnp.float32),
            buf((CHUNK, C), jnp.float32),
            buf((CHUNK, C), jnp.float32),
            buf((CHUNK, C), jnp.float32),
            buf((C, CHUNK + 1), jnp.float32),
            [pltpu.SemaphoreType.DMA for _ in range(2)],
        ],
    )
    def k(table_hbm, gx_hbm, gy_hbm, out_hbm,
          gxv, gyv, i00, i01, i10, i11, w00, w01, w10, w11,
          r00, r01, r10, r11, outv, sem):
        wid = lax.axis_index("s") * 2 + lax.axis_index("c")
        base = wid * BPW
        nidx = wid // WPB
        nbase = nidx * HW
        qbase = (wid % WPB) * BPW

        def stage(cur, s):
            off = base + cur * CHUNK
            pltpu.sync_copy(gx_hbm.at[pl.ds(off, CHUNK)], gxv[s])
            pltpu.sync_copy(gy_hbm.at[pl.ds(off, CHUNK)], gyv[s])

            def grp_body(j, c2):
                sl = pl.ds(j * LANES, LANES)
                x = gxv[s][sl]
                y = gyv[s][sl]
                ix = (x + 1.0) * (W * 0.5) - 0.5
                iy = (y + 1.0) * (H * 0.5) - 0.5
                x0 = _floor_i32(ix)
                y0 = _floor_i32(iy)
                fx = ix - x0.astype(jnp.float32)
                fy = iy - y0.astype(jnp.float32)
                vx0 = (x0 >= 0) & (x0 <= W - 1)
                vx1 = (x0 >= -1) & (x0 <= W - 2)
                vy0 = (y0 >= 0) & (y0 <= H - 1)
                vy1 = (y0 >= -1) & (y0 <= H - 2)
                zero = jnp.zeros((LANES,), jnp.float32)
                wx0 = jnp.where(vx0, 1.0 - fx, zero)
                wx1 = jnp.where(vx1, fx, zero)
                wy0 = jnp.where(vy0, 1.0 - fy, zero)
                wy1 = jnp.where(vy1, fy, zero)
                xc0 = jnp.clip(x0, 0, W - 1)
                xc1 = jnp.clip(x0 + 1, 0, W - 1)
                yr0 = nbase + jnp.clip(y0, 0, H - 1) * W
                yr1 = nbase + jnp.clip(y0 + 1, 0, H - 1) * W
                i00[s][sl] = yr0 + xc0
                i01[s][sl] = yr0 + xc1
                i10[s][sl] = yr1 + xc0
                i11[s][sl] = yr1 + xc1
                w00[s][sl] = wy0 * wx0
                w01[s][sl] = wy0 * wx1
                w10[s][sl] = wy1 * wx0
                w11[s][sl] = wy1 * wx1
                return c2

            lax.fori_loop(0, GRPS, grp_body, 0)

            pltpu.async_copy(table_hbm.at[i00[s]], r00[s], sem[s])
            pltpu.async_copy(table_hbm.at[i01[s]], r01[s], sem[s])
            pltpu.async_copy(table_hbm.at[i10[s]], r10[s], sem[s])
            pltpu.async_copy(table_hbm.at[i11[s]], r11[s], sem[s])

        def drain(s):
            pltpu.make_async_copy(table_hbm.at[i00[s]], r00[s], sem[s]).wait()
            pltpu.make_async_copy(table_hbm.at[i01[s]], r01[s], sem[s]).wait()
            pltpu.make_async_copy(table_hbm.at[i10[s]], r10[s], sem[s]).wait()
            pltpu.make_async_copy(table_hbm.at[i11[s]], r11[s], sem[s]).wait()

        def interp(cur, s):
            q = qbase + cur * CHUNK
            hrow = q // W
            wcol = q % W
            lanes = lax.iota(jnp.int32, LANES)

            def grp_body2(g, c2):
                gs = pl.ds(g * LANES, LANES)
                wv00 = w00[s][gs]
                wv01 = w01[s][gs]
                wv10 = w10[s][gs]
                wv11 = w11[s][gs]
                for p in range(LANES):
                    pidx = jnp.full((LANES,), p, jnp.int32)
                    s00 = jnp.take_along_axis(
                        wv00, pidx, axis=0, mode="promise_in_bounds")
                    s01 = jnp.take_along_axis(
                        wv01, pidx, axis=0, mode="promise_in_bounds")
                    s10 = jnp.take_along_axis(
                        wv10, pidx, axis=0, mode="promise_in_bounds")
                    s11 = jnp.take_along_axis(
                        wv11, pidx, axis=0, mode="promise_in_bounds")
                    i = g * LANES + p
                    widx = lax.broadcast(i, (LANES,))
                    for r in range(CV):
                        cs = r * LANES
                        a = r00[s][i, pl.ds(cs, LANES)] * s00
                        a = a + r01[s][i, pl.ds(cs, LANES)] * s01
                        a = a + r10[s][i, pl.ds(cs, LANES)] * s10
                        a = a + r11[s][i, pl.ds(cs, LANES)] * s11
                        plsc.store_scatter(outv[s], [lanes + cs, widx], a)
                return c2

            lax.fori_loop(0, GRPS, grp_body2, 0)
            pltpu.sync_copy(outv[s].at[:, pl.ds(0, CHUNK)],
                            out_hbm.at[nidx, :, hrow, pl.ds(wcol, CHUNK)])

        stage(0, 0)

        @pl.loop(0, NCHUNK, step=2)
        def _(g):
            for b in (0, 1):
                cur = g + b

                @pl.when(cur + 1 < NCHUNK)
                def _():
                    stage(cur + 1, 1 - b)

                drain(b)
                interp(cur, b)

    return k(table, gx, gy)


def kernel(input, grid):
    table = _sc_transpose(input)
    gx = grid[..., 0].reshape(B)
    gy = grid[..., 1].reshape(B)
    return _sc_grid_sample(table, gx, gy)


# speedup vs baseline: 1.5336x; 1.2115x over previous
import functools

import jax
import jax.numpy as jnp
from jax import lax
from jax.experimental import pallas as pl
from jax.experimental.pallas import tpu as pltpu
from jax.experimental.pallas import tpu_sc as plsc

N, C, H, W = 4, 96, 224, 224
HW = H * W
B = N * HW
NW = 32
BPW = B // NW
CHUNK = 112
NCHUNK = BPW // CHUNK
LANES = 16
GRPS = CHUNK // LANES
CV = C // LANES
WPB = HW // BPW


def _floor_i32(v):
    t = v.astype(jnp.int32)
    return t - jnp.where(t.astype(jnp.float32) > v, 1, 0)


ROWS = N * H // NW


def _sc_transpose(inp):
    mesh = plsc.VectorSubcoreMesh(core_axis_name="c", subcore_axis_name="s")

    buf = lambda shape, dt: [pltpu.VMEM(shape, dt) for _ in range(2)]

    @functools.partial(
        pl.kernel,
        mesh=mesh,
        out_type=jax.ShapeDtypeStruct((B, C), jnp.float32),
        compiler_params=pltpu.CompilerParams(
            needs_layout_passes=False, use_tc_tiling_on_sc=False),
        scratch_types=[
            buf((C, W + 1), jnp.float32),
            buf((W, C), jnp.float32),
            [pltpu.SemaphoreType.DMA for _ in range(2)],
            [pltpu.SemaphoreType.DMA for _ in range(2)],
        ],
    )
    def kt(inp_hbm, table_hbm, inbuf, tbuf, semi, semo):
        wid = lax.axis_index("s") * 2 + lax.axis_index("c")
        rowbase = wid * ROWS
        lanes = lax.iota(jnp.int32, LANES)

        def issue_in(j, s):
            gr = rowbase + j
            n = gr // H
            h = gr % H
            pltpu.async_copy(
                inp_hbm.at[n, :, h, :], inbuf[s].at[:, pl.ds(0, W)], semi[s])

        def drain_in(s):
            pltpu.make_async_copy(
                inp_hbm.at[0, :, 0, :], inbuf[s].at[:, pl.ds(0, W)],
                semi[s]).wait()

        def transpose(s):
            @plsc.parallel_loop(0, W, unroll=4)
            def _(pix):
                widx = lax.broadcast(pix, (LANES,))
                for c0 in range(CV):
                    v = plsc.load_gather(inbuf[s], [lanes + c0 * LANES, widx])
                    tbuf[s][pix, pl.ds(c0 * LANES, LANES)] = v

        def issue_out(j, s):
            gr = rowbase + j
            pltpu.async_copy(tbuf[s], table_hbm.at[pl.ds(gr * W, W)], semo[s])

        def drain_out(s):
            pltpu.make_async_copy(
                tbuf[s], table_hbm.at[pl.ds(0, W)], semo[s]).wait()

        issue_in(0, 0)

        @pl.loop(0, ROWS, step=2)
        def _(g):
            for b in (0, 1):
                cur = g + b

                @pl.when(cur + 1 < ROWS)
                def _():
                    issue_in(cur + 1, 1 - b)

                drain_in(b)

                @pl.when(cur >= 2)
                def _():
                    drain_out(b)

                transpose(b)
                issue_out(cur, b)

        drain_out(0)
        drain_out(1)

    return kt(inp)


def _sc_grid_sample(table, gx, gy):
    mesh = plsc.VectorSubcoreMesh(core_axis_name="c", subcore_axis_name="s")

    buf = lambda shape, dt: [pltpu.VMEM(shape, dt) for _ in range(2)]

    @functools.partial(
        pl.kernel,
        mesh=mesh,
        out_type=jax.ShapeDtypeStruct((N, C, H, W), jnp.float32),
        compiler_params=pltpu.CompilerParams(
            needs_layout_passes=False, use_tc_tiling_on_sc=False),
        scratch_types=[
            buf((CHUNK,), jnp.float32),
            buf((CHUNK,), jnp.float32),
            buf((CHUNK,), jnp.int32),
            buf((CHUNK,), jnp.int32),
            buf((CHUNK,), jnp.int32),
            buf((CHUNK,), jnp.int32),
            buf((CHUNK,), jnp.float32),
            buf((CHUNK,), jnp.float32),
            buf((CHUNK,), jnp.float32),
            buf((CHUNK,), jnp.float32),
            buf((CHUNK, C), jnp.float32),
            buf((CHUNK, C), jnp.float32),
            buf((CHUNK, C), jnp.float32),
            buf((CHUNK, C), jnp.float32),
            buf((C, CHUNK + 1), jnp.float32),
            [pltpu.SemaphoreType.DMA for _ in range(2)],
        ],
    )
    def k(table_hbm, gx_hbm, gy_hbm, out_hbm,
          gxv, gyv, i00, i01, i10, i11, w00, w01, w10, w11,
          r00, r01, r10, r11, outv, sem):
        wid = lax.axis_index("s") * 2 + lax.axis_index("c")
        base = wid * BPW
        nidx = wid // WPB
        nbase = nidx * HW
        qbase = (wid % WPB) * BPW

        def stage(cur, s):
            off = base + cur * CHUNK
            pltpu.sync_copy(gx_hbm.at[pl.ds(off, CHUNK)], gxv[s])
            pltpu.sync_copy(gy_hbm.at[pl.ds(off, CHUNK)], gyv[s])

            @plsc.parallel_loop(0, GRPS, unroll=1)
            def grp_body(j):
                sl = pl.ds(j * LANES, LANES)
                x = gxv[s][sl]
                y = gyv[s][sl]
                ix = (x + 1.0) * (W * 0.5) - 0.5
                iy = (y + 1.0) * (H * 0.5) - 0.5
                x0 = _floor_i32(ix)
                y0 = _floor_i32(iy)
                fx = ix - x0.astype(jnp.float32)
                fy = iy - y0.astype(jnp.float32)
                vx0 = (x0 >= 0) & (x0 <= W - 1)
                vx1 = (x0 >= -1) & (x0 <= W - 2)
                vy0 = (y0 >= 0) & (y0 <= H - 1)
                vy1 = (y0 >= -1) & (y0 <= H - 2)
                zero = jnp.zeros((LANES,), jnp.float32)
                wx0 = jnp.where(vx0, 1.0 - fx, zero)
                wx1 = jnp.where(vx1, fx, zero)
                wy0 = jnp.where(vy0, 1.0 - fy, zero)
                wy1 = jnp.where(vy1, fy, zero)
                xc0 = jnp.clip(x0, 0, W - 1)
                xc1 = jnp.clip(x0 + 1, 0, W - 1)
                yr0 = nbase + jnp.clip(y0, 0, H - 1) * W
                yr1 = nbase + jnp.clip(y0 + 1, 0, H - 1) * W
                i00[s][sl] = yr0 + xc0
                i01[s][sl] = yr0 + xc1
                i10[s][sl] = yr1 + xc0
                i11[s][sl] = yr1 + xc1
                w00[s][sl] = wy0 * wx0
                w01[s][sl] = wy0 * wx1
                w10[s][sl] = wy1 * wx0
                w11[s][sl] = wy1 * wx1

            pltpu.async_copy(table_hbm.at[i00[s]], r00[s], sem[s])
            pltpu.async_copy(table_hbm.at[i01[s]], r01[s], sem[s])
            pltpu.async_copy(table_hbm.at[i10[s]], r10[s], sem[s])
            pltpu.async_copy(table_hbm.at[i11[s]], r11[s], sem[s])

        def drain(s):
            pltpu.make_async_copy(table_hbm.at[i00[s]], r00[s], sem[s]).wait()
            pltpu.make_async_copy(table_hbm.at[i01[s]], r01[s], sem[s]).wait()
            pltpu.make_async_copy(table_hbm.at[i10[s]], r10[s], sem[s]).wait()
            pltpu.make_async_copy(table_hbm.at[i11[s]], r11[s], sem[s]).wait()

        def interp(cur, s):
            q = qbase + cur * CHUNK
            hrow = q // W
            wcol = q % W
            lanes = lax.iota(jnp.int32, LANES)

            @plsc.parallel_loop(0, GRPS, unroll=1)
            def grp_body2(g):
                gs = pl.ds(g * LANES, LANES)
                wv00 = w00[s][gs]
                wv01 = w01[s][gs]
                wv10 = w10[s][gs]
                wv11 = w11[s][gs]
                for p in range(LANES):
                    pidx = jnp.full((LANES,), p, jnp.int32)
                    s00 = jnp.take_along_axis(
                        wv00, pidx, axis=0, mode="promise_in_bounds")
                    s01 = jnp.take_along_axis(
                        wv01, pidx, axis=0, mode="promise_in_bounds")
                    s10 = jnp.take_along_axis(
                        wv10, pidx, axis=0, mode="promise_in_bounds")
                    s11 = jnp.take_along_axis(
                        wv11, pidx, axis=0, mode="promise_in_bounds")
                    i = g * LANES + p
                    widx = lax.broadcast(i, (LANES,))
                    for r in range(CV):
                        cs = r * LANES
                        a0 = r00[s][i, pl.ds(cs, LANES)] * s00
                        a1 = r01[s][i, pl.ds(cs, LANES)] * s01
                        a2 = r10[s][i, pl.ds(cs, LANES)] * s10
                        a3 = r11[s][i, pl.ds(cs, LANES)] * s11
                        plsc.store_scatter(
                            outv[s], [lanes + cs, widx], (a0 + a1) + (a2 + a3))
            pltpu.sync_copy(outv[s].at[:, pl.ds(0, CHUNK)],
                            out_hbm.at[nidx, :, hrow, pl.ds(wcol, CHUNK)])

        stage(0, 0)

        @pl.loop(0, NCHUNK, step=2)
        def _(g):
            for b in (0, 1):
                cur = g + b

                @pl.when(cur + 1 < NCHUNK)
                def _():
                    stage(cur + 1, 1 - b)

                drain(b)
                interp(cur, b)

    return k(table, gx, gy)


def kernel(input, grid):
    table = _sc_transpose(input)
    gx = grid[..., 0].reshape(B)
    gy = grid[..., 1].reshape(B)
    return _sc_grid_sample(table, gx, gy)


# speedup vs baseline: 2.3633x; 1.5410x over previous
import functools

import jax
import jax.numpy as jnp
from jax import lax
from jax.experimental import pallas as pl
from jax.experimental.pallas import tpu as pltpu
from jax.experimental.pallas import tpu_sc as plsc

N, C, H, W = 4, 96, 224, 224
HW = H * W
B = N * HW
NW = 32
BPW = B // NW
CHUNK = 112
NCHUNK = BPW // CHUNK
LANES = 16
GRPS = CHUNK // LANES
CV = C // LANES
WPB = HW // BPW


def _floor_i32(v):
    t = v.astype(jnp.int32)
    return t - jnp.where(t.astype(jnp.float32) > v, 1, 0)


ROWS = N * H // NW


def _sc_transpose(inp):
    mesh = plsc.VectorSubcoreMesh(core_axis_name="c", subcore_axis_name="s")

    buf = lambda shape, dt: [pltpu.VMEM(shape, dt) for _ in range(2)]

    @functools.partial(
        pl.kernel,
        mesh=mesh,
        out_type=jax.ShapeDtypeStruct((B, C), jnp.float32),
        compiler_params=pltpu.CompilerParams(
            needs_layout_passes=False, use_tc_tiling_on_sc=False),
        scratch_types=[
            buf((C, W + 1), jnp.float32),
            buf((W, C), jnp.float32),
            [pltpu.SemaphoreType.DMA for _ in range(2)],
            [pltpu.SemaphoreType.DMA for _ in range(2)],
        ],
    )
    def kt(inp_hbm, table_hbm, inbuf, tbuf, semi, semo):
        wid = lax.axis_index("s") * 2 + lax.axis_index("c")
        rowbase = wid * ROWS
        lanes = lax.iota(jnp.int32, LANES)

        def issue_in(j, s):
            gr = rowbase + j
            n = gr // H
            h = gr % H
            pltpu.async_copy(
                inp_hbm.at[n, :, h, :], inbuf[s].at[:, pl.ds(0, W)], semi[s])

        def drain_in(s):
            pltpu.make_async_copy(
                inp_hbm.at[0, :, 0, :], inbuf[s].at[:, pl.ds(0, W)],
                semi[s]).wait()

        def transpose(s):
            @plsc.parallel_loop(0, W, unroll=4)
            def _(pix):
                widx = lax.broadcast(pix, (LANES,))
                for c0 in range(CV):
                    v = plsc.load_gather(inbuf[s], [lanes + c0 * LANES, widx])
                    tbuf[s][pix, pl.ds(c0 * LANES, LANES)] = v

        def issue_out(j, s):
            gr = rowbase + j
            pltpu.async_copy(tbuf[s], table_hbm.at[pl.ds(gr * W, W)], semo[s])

        def drain_out(s):
            pltpu.make_async_copy(
                tbuf[s], table_hbm.at[pl.ds(0, W)], semo[s]).wait()

        issue_in(0, 0)

        @pl.loop(0, ROWS, step=2)
        def _(g):
            for b in (0, 1):
                cur = g + b

                @pl.when(cur + 1 < ROWS)
                def _():
                    issue_in(cur + 1, 1 - b)

                drain_in(b)

                @pl.when(cur >= 2)
                def _():
                    drain_out(b)

                transpose(b)
                issue_out(cur, b)

        drain_out(0)
        drain_out(1)

    return kt(inp)


def _sc_grid_sample(table, gx, gy):
    mesh = plsc.VectorSubcoreMesh(core_axis_name="c", subcore_axis_name="s")

    buf = lambda shape, dt: [pltpu.VMEM(shape, dt) for _ in range(2)]

    @functools.partial(
        pl.kernel,
        mesh=mesh,
        out_type=jax.ShapeDtypeStruct((N, C, H, W), jnp.float32),
        compiler_params=pltpu.CompilerParams(
            needs_layout_passes=False, use_tc_tiling_on_sc=False),
        scratch_types=[
            buf((CHUNK,), jnp.float32),
            buf((CHUNK,), jnp.float32),
            buf((CHUNK,), jnp.int32),
            buf((CHUNK,), jnp.int32),
            buf((CHUNK,), jnp.int32),
            buf((CHUNK,), jnp.int32),
            buf((CHUNK,), jnp.float32),
            buf((CHUNK,), jnp.float32),
            buf((CHUNK,), jnp.float32),
            buf((CHUNK,), jnp.float32),
            buf((CHUNK, C), jnp.float32),
            buf((CHUNK, C), jnp.float32),
            buf((CHUNK, C), jnp.float32),
            buf((CHUNK, C), jnp.float32),
            buf((C, CHUNK + 1), jnp.float32),
            [pltpu.SemaphoreType.DMA for _ in range(2)],
        ],
    )
    def k(table_hbm, gx_hbm, gy_hbm, out_hbm,
          gxv, gyv, i00, i01, i10, i11, w00, w01, w10, w11,
          r00, r01, r10, r11, outv, sem):
        wid = lax.axis_index("s") * 2 + lax.axis_index("c")
        base = wid * BPW
        nidx = wid // WPB
        nbase = nidx * HW
        qbase = (wid % WPB) * BPW

        def stage(cur, s):
            off = base + cur * CHUNK
            pltpu.sync_copy(gx_hbm.at[pl.ds(off, CHUNK)], gxv[s])
            pltpu.sync_copy(gy_hbm.at[pl.ds(off, CHUNK)], gyv[s])

            @plsc.parallel_loop(0, GRPS, unroll=1)
            def grp_body(j):
                sl = pl.ds(j * LANES, LANES)
                x = gxv[s][sl]
                y = gyv[s][sl]
                ix = (x + 1.0) * (W * 0.5) - 0.5
                iy = (y + 1.0) * (H * 0.5) - 0.5
                x0 = _floor_i32(ix)
                y0 = _floor_i32(iy)
                fx = ix - x0.astype(jnp.float32)
                fy = iy - y0.astype(jnp.float32)
                vx0 = (x0 >= 0) & (x0 <= W - 1)
                vx1 = (x0 >= -1) & (x0 <= W - 2)
                vy0 = (y0 >= 0) & (y0 <= H - 1)
                vy1 = (y0 >= -1) & (y0 <= H - 2)
                zero = jnp.zeros((LANES,), jnp.float32)
                wx0 = jnp.where(vx0, 1.0 - fx, zero)
                wx1 = jnp.where(vx1, fx, zero)
                wy0 = jnp.where(vy0, 1.0 - fy, zero)
                wy1 = jnp.where(vy1, fy, zero)
                xc0 = jnp.clip(x0, 0, W - 1)
                xc1 = jnp.clip(x0 + 1, 0, W - 1)
                yr0 = nbase + jnp.clip(y0, 0, H - 1) * W
                yr1 = nbase + jnp.clip(y0 + 1, 0, H - 1) * W
                i00[s][sl] = yr0 + xc0
                i01[s][sl] = yr0 + xc1
                i10[s][sl] = yr1 + xc0
                i11[s][sl] = yr1 + xc1
                w00[s][sl] = wy0 * wx0
                w01[s][sl] = wy0 * wx1
                w10[s][sl] = wy1 * wx0
                w11[s][sl] = wy1 * wx1

            pltpu.async_copy(table_hbm.at[i00[s]], r00[s], sem[s])
            pltpu.async_copy(table_hbm.at[i01[s]], r01[s], sem[s])
            pltpu.async_copy(table_hbm.at[i10[s]], r10[s], sem[s])
            pltpu.async_copy(table_hbm.at[i11[s]], r11[s], sem[s])

        def drain(s):
            pltpu.make_async_copy(table_hbm.at[i00[s]], r00[s], sem[s]).wait()
            pltpu.make_async_copy(table_hbm.at[i01[s]], r01[s], sem[s]).wait()
            pltpu.make_async_copy(table_hbm.at[i10[s]], r10[s], sem[s]).wait()
            pltpu.make_async_copy(table_hbm.at[i11[s]], r11[s], sem[s]).wait()

        def interp(cur, s):
            q = qbase + cur * CHUNK
            hrow = q // W
            wcol = q % W
            lanes = lax.iota(jnp.int32, LANES)
            lcs = [lanes + r * LANES for r in range(CV)]

            @plsc.parallel_loop(0, GRPS, unroll=1)
            def grp_body2(g):
                gs = pl.ds(g * LANES, LANES)
                wv00 = w00[s][gs]
                wv01 = w01[s][gs]
                wv10 = w10[s][gs]
                wv11 = w11[s][gs]
                gbase = g * LANES

                @plsc.parallel_loop(0, LANES, unroll=4)
                def pix_body(p):
                    pidx = lax.broadcast(p, (LANES,))
                    s00 = jnp.take_along_axis(
                        wv00, pidx, axis=0, mode="promise_in_bounds")
                    s01 = jnp.take_along_axis(
                        wv01, pidx, axis=0, mode="promise_in_bounds")
                    s10 = jnp.take_along_axis(
                        wv10, pidx, axis=0, mode="promise_in_bounds")
                    s11 = jnp.take_along_axis(
                        wv11, pidx, axis=0, mode="promise_in_bounds")
                    i = gbase + p
                    widx = pidx + gbase
                    for r in range(CV):
                        cs = r * LANES
                        a0 = r00[s][i, pl.ds(cs, LANES)] * s00
                        a1 = r01[s][i, pl.ds(cs, LANES)] * s01
                        a2 = r10[s][i, pl.ds(cs, LANES)] * s10
                        a3 = r11[s][i, pl.ds(cs, LANES)] * s11
                        plsc.store_scatter(
                            outv[s], [lcs[r], widx], (a0 + a1) + (a2 + a3))
            pltpu.sync_copy(outv[s].at[:, pl.ds(0, CHUNK)],
                            out_hbm.at[nidx, :, hrow, pl.ds(wcol, CHUNK)])

        stage(0, 0)

        @pl.loop(0, NCHUNK, step=2)
        def _(g):
            for b in (0, 1):
                cur = g + b

                @pl.when(cur + 1 < NCHUNK)
                def _():
                    stage(cur + 1, 1 - b)

                drain(b)
                interp(cur, b)

    return k(table, gx, gy)


def kernel(input, grid):
    table = _sc_transpose(input)
    gx = grid[..., 0].reshape(B)
    gy = grid[..., 1].reshape(B)
    return _sc_grid_sample(table, gx, gy)
